# Initial kernel scaffold; baseline (speedup 1.0000x reference)
import functools

import jax
import jax.numpy as jnp
from jax import lax
from jax.experimental import pallas as pl
from jax.experimental.pallas import tpu as pltpu
from jax.experimental.pallas import tpu_sc as plsc

N = 10000
E = 320000
D = 128
NC = 2
NS = 16
NW = NC * NS
CHUNK = 128
CPT = 79
EPT = CPT * CHUNK
EPAD = EPT * NW
NPAD = 10016
RPT = NPAD // NS

_mesh = plsc.VectorSubcoreMesh(core_axis_name="c", subcore_axis_name="s")


@functools.partial(
    pl.kernel,
    out_type=jax.ShapeDtypeStruct((NW, NPAD), jnp.float32),
    mesh=_mesh,
    scratch_types=[
        pltpu.VMEM((CPT, CHUNK), jnp.int32),
        pltpu.VMEM((NPAD,), jnp.float32),
    ],
)
def _deg_kernel(dst_hbm, out_hbm, idx_v, cnt_v):
    wid = lax.axis_index("s") * NC + lax.axis_index("c")
    zero16 = jnp.zeros((16,), jnp.float32)

    @pl.loop(0, NPAD, step=16)
    def _(i):
        cnt_v[pl.ds(i, 16)] = zero16

    pltpu.sync_copy(dst_hbm.at[pl.ds(wid * CPT, CPT)], idx_v)
    ones16 = jnp.ones((16,), jnp.float32)

    @pl.loop(0, CPT)
    def _(j):
        @pl.loop(0, CHUNK, step=16)
        def _(k):
            plsc.addupdate_scatter(cnt_v, [idx_v[j, pl.ds(k, 16)]], ones16)

    pltpu.sync_copy(cnt_v, out_hbm.at[wid])


@functools.partial(
    pl.kernel,
    out_type=jax.ShapeDtypeStruct((NC, NPAD, D), jnp.float32),
    mesh=_mesh,
    scratch_types=[
        pltpu.VMEM((CPT, CHUNK), jnp.int32),
        pltpu.VMEM((CPT, CHUNK), jnp.int32),
        pltpu.VMEM((CHUNK, D), jnp.float32),
        pltpu.VMEM((CHUNK, D), jnp.float32),
        pltpu.VMEM_SHARED((NPAD, D), jnp.float32),
        pltpu.SemaphoreType.DMA,
        pltpu.SemaphoreType.DMA,
    ],
)
def _edge_kernel(src_hbm, dst_hbm, node_hbm, out_hbm,
                 src_v, dst_v, rows0, rows1, acc, sem0, sem1):
    c = lax.axis_index("c")
    s = lax.axis_index("s")
    wid = s * NC + c
    zero16 = jnp.zeros((16,), jnp.float32)

    @pl.loop(0, CHUNK)
    def _(r):
        @pl.loop(0, D, step=16)
        def _(col):
            rows0[r, pl.ds(col, 16)] = zero16

    base = s * RPT
    for t in range(4):
        pltpu.sync_copy(rows0, acc.at[pl.ds(base + t * CHUNK, CHUNK)])
    pltpu.sync_copy(rows0.at[pl.ds(0, RPT - 4 * CHUNK)],
                    acc.at[pl.ds(base + 4 * CHUNK, RPT - 4 * CHUNK)])

    pltpu.sync_copy(src_hbm.at[pl.ds(wid * CPT, CPT)], src_v)
    pltpu.sync_copy(dst_hbm.at[pl.ds(wid * CPT, CPT)], dst_v)

    plsc.subcore_barrier()

    pltpu.make_async_copy(node_hbm.at[src_v.at[0]], rows0, sem0).start()

    @pl.loop(0, (CPT - 1) // 2)
    def _(k):
        j = 2 * k
        pltpu.make_async_copy(node_hbm.at[src_v.at[j]], rows0, sem0).wait()
        pltpu.make_async_copy(node_hbm.at[src_v.at[j + 1]], rows1, sem1).start()
        pltpu.sync_copy(rows0, acc.at[dst_v.at[j]], add=True)
        pltpu.make_async_copy(node_hbm.at[src_v.at[j + 1]], rows1, sem1).wait()
        pltpu.make_async_copy(node_hbm.at[src_v.at[j + 2]], rows0, sem0).start()
        pltpu.sync_copy(rows1, acc.at[dst_v.at[j + 1]], add=True)

    pltpu.make_async_copy(node_hbm.at[src_v.at[CPT - 1]], rows0, sem0).wait()
    pltpu.sync_copy(rows0, acc.at[dst_v.at[CPT - 1]], add=True)

    plsc.subcore_barrier()
    pltpu.sync_copy(acc.at[pl.ds(base, RPT)], out_hbm.at[c, pl.ds(base, RPT)])


_BN = 1000


def _scale_body(p_ref, x_ref, node_ref, s_ref):
    deg = jnp.sum(p_ref[...], axis=0)
    s = 1.0 / jnp.sqrt(jnp.maximum(deg, 1.0))
    s_ref[...] = s[None, :]
    node_ref[...] = x_ref[...] * s[:, None]


def _scale_call(partials, x):
    return pl.pallas_call(
        _scale_body,
        grid=(N // _BN,),
        in_specs=[
            pl.BlockSpec((NW, _BN), lambda i: (0, i)),
            pl.BlockSpec((_BN, D), lambda i: (i, 0)),
        ],
        out_specs=[
            pl.BlockSpec((_BN, D), lambda i: (i, 0)),
            pl.BlockSpec((1, _BN), lambda i: (0, i)),
        ],
        out_shape=[
            jax.ShapeDtypeStruct((N, D), jnp.float32),
            jax.ShapeDtypeStruct((1, N), jnp.float32),
        ],
    )(partials, x)


def _final_body(x_ref, s_ref, p_ref, w_ref, b_ref, o_ref):
    agg = p_ref[0] + p_ref[1]
    h = x_ref[...] + agg * s_ref[0][:, None]
    z = lax.dot_general(h, w_ref[...], (((1,), (1,)), ((), ())),
                        preferred_element_type=jnp.float32) + b_ref[...]
    o_ref[...] = jnp.where(z >= 0, z, 0.01 * z)


def _final_call(x, s, pagg, W, b2):
    return pl.pallas_call(
        _final_body,
        grid=(N // _BN,),
        in_specs=[
            pl.BlockSpec((_BN, D), lambda i: (i, 0)),
            pl.BlockSpec((1, _BN), lambda i: (0, i)),
            pl.BlockSpec((NC, _BN, D), lambda i: (0, i, 0)),
            pl.BlockSpec((D, D), lambda i: (0, 0)),
            pl.BlockSpec((1, D), lambda i: (0, 0)),
        ],
        out_specs=pl.BlockSpec((_BN, D), lambda i: (i, 0)),
        out_shape=jax.ShapeDtypeStruct((N, D), jnp.float32),
    )(x, s, pagg, W, b2)


def kernel(entity_embed, edge_index, W, b):
    src = edge_index[0]
    dst = edge_index[1]
    src2d = jnp.concatenate(
        [src, jnp.zeros((EPAD - E,), jnp.int32)]).reshape(NW * CPT, CHUNK)
    dst2d = jnp.concatenate(
        [dst, jnp.full((EPAD - E,), N, jnp.int32)]).reshape(NW * CPT, CHUNK)
    partials = _deg_kernel(dst2d)
    node, s = _scale_call(partials[:, :N], entity_embed)
    pagg = _edge_kernel(src2d, dst2d, node)
    return _final_call(entity_embed, s, pagg, W, b.reshape(1, D))


# speedup vs baseline: 3.9324x; 3.9324x over previous
import dataclasses
import functools

import jax
import jax.numpy as jnp
from jax import lax
from jax.experimental import pallas as pl
from jax.experimental.pallas import tpu as pltpu
from jax.experimental.pallas import tpu_sc as plsc

N = 10000
E = 320000
D = 128
NC = 2
NS = 16
NW = NC * NS
CHUNK = 128
BLK = 8
CPT = 80
NBLK = CPT // BLK
EPT = CPT * CHUNK
EPAD = EPT * NW
NPAD = 10112
RPT = NPAD // NS

_mesh = plsc.VectorSubcoreMesh(core_axis_name="c", subcore_axis_name="s")

_sc_params = pltpu.CompilerParams()
if "needs_layout_passes" in pltpu.CompilerParams.__dataclass_fields__:
    _sc_params = dataclasses.replace(_sc_params, needs_layout_passes=False)


@functools.partial(
    pl.kernel,
    out_type=jax.ShapeDtypeStruct((NW, NPAD), jnp.float32),
    mesh=_mesh,
    scratch_types=[
        pltpu.VMEM((CPT, CHUNK), jnp.int32),
        pltpu.VMEM((NPAD,), jnp.float32),
    ],
    compiler_params=_sc_params,
)
def _deg_kernel(dst_hbm, out_hbm, idx_v, cnt_v):
    wid = lax.axis_index("s") * NC + lax.axis_index("c")
    zero16 = jnp.zeros((16,), jnp.float32)

    @pl.loop(0, NPAD, step=16)
    def _(i):
        cnt_v[pl.ds(i, 16)] = zero16

    pltpu.sync_copy(dst_hbm.at[wid], idx_v)
    ones16 = jnp.ones((16,), jnp.float32)

    @pl.loop(0, CPT)
    def _(j):
        @pl.loop(0, CHUNK, step=16)
        def _(k):
            plsc.addupdate_scatter(cnt_v, [idx_v[j, pl.ds(k, 16)]], ones16)

    pltpu.sync_copy(cnt_v, out_hbm.at[wid])


@functools.partial(
    pl.kernel,
    out_type=jax.ShapeDtypeStruct((NC, NPAD, D), jnp.float32),
    mesh=_mesh,
    scratch_types=[
        pltpu.VMEM((BLK, CHUNK), jnp.int32),
        pltpu.VMEM((BLK, CHUNK), jnp.int32),
        pltpu.VMEM((CHUNK, D), jnp.float32),
        pltpu.VMEM((CHUNK, D), jnp.float32),
        pltpu.VMEM_SHARED((NPAD, D), jnp.float32),
        pltpu.SemaphoreType.DMA,
        pltpu.SemaphoreType.DMA,
    ],
    compiler_params=_sc_params,
)
def _edge_kernel(src_hbm, dst_hbm, node_hbm, out_hbm,
                 src_v, dst_v, rows0, rows1, acc, sem0, sem1):
    c = lax.axis_index("c")
    s = lax.axis_index("s")
    wid = s * NC + c
    zero16 = jnp.zeros((16,), jnp.float32)

    @pl.loop(0, CHUNK)
    def _(r):
        @pl.loop(0, D, step=16)
        def _(col):
            rows0[r, pl.ds(col, 16)] = zero16

    base = s * RPT
    for t in range(RPT // CHUNK):
        pltpu.sync_copy(rows0, acc.at[pl.ds(base + t * CHUNK, CHUNK)])
    if RPT % CHUNK:
        pltpu.sync_copy(rows0.at[pl.ds(0, RPT % CHUNK)],
                        acc.at[pl.ds(base + (RPT // CHUNK) * CHUNK, RPT % CHUNK)])

    plsc.subcore_barrier()

    @pl.loop(0, NBLK)
    def _(blk):
        pltpu.sync_copy(src_hbm.at[wid, pl.ds(blk * BLK, BLK)], src_v)
        pltpu.sync_copy(dst_hbm.at[wid, pl.ds(blk * BLK, BLK)], dst_v)
        pltpu.make_async_copy(node_hbm.at[src_v.at[0]], rows0, sem0).start()
        for jj in range(BLK):
            buf, sem = (rows0, sem0) if jj % 2 == 0 else (rows1, sem1)
            nbuf, nsem = (rows1, sem1) if jj % 2 == 0 else (rows0, sem0)
            pltpu.make_async_copy(node_hbm.at[src_v.at[jj]], buf, sem).wait()
            if jj + 1 < BLK:
                pltpu.make_async_copy(
                    node_hbm.at[src_v.at[jj + 1]], nbuf, nsem).start()
            pltpu.sync_copy(buf, acc.at[dst_v.at[jj]], add=True)

    plsc.subcore_barrier()
    pltpu.sync_copy(acc.at[pl.ds(base, RPT)], out_hbm.at[c, pl.ds(base, RPT)])


_BN = 1000


def _scale_body(p_ref, x_ref, node_ref, s_ref):
    deg = jnp.sum(p_ref[...], axis=0)
    s = 1.0 / jnp.sqrt(jnp.maximum(deg, 1.0))
    s_ref[...] = s[:, None]
    node_ref[...] = x_ref[...] * s[:, None]


def _scale_call(partials, x):
    return pl.pallas_call(
        _scale_body,
        out_shape=[
            jax.ShapeDtypeStruct((N, D), jnp.float32),
            jax.ShapeDtypeStruct((N, 1), jnp.float32),
        ],
    )(partials, x)


def _final_body(x_ref, s_ref, p_ref, w_ref, b_ref, o_ref):
    agg = p_ref[0] + p_ref[1]
    h = x_ref[...] + agg * s_ref[...]
    z = lax.dot_general(h, w_ref[...], (((1,), (1,)), ((), ())),
                        preferred_element_type=jnp.float32) + b_ref[...]
    o_ref[...] = jnp.where(z >= 0, z, 0.01 * z)


def _final_call(x, s, pagg, W, b2):
    return pl.pallas_call(
        _final_body,
        grid=(N // _BN,),
        in_specs=[
            pl.BlockSpec((_BN, D), lambda i: (i, 0)),
            pl.BlockSpec((_BN, 1), lambda i: (i, 0)),
            pl.BlockSpec((NC, _BN, D), lambda i: (0, i, 0)),
            pl.BlockSpec((D, D), lambda i: (0, 0)),
            pl.BlockSpec((1, D), lambda i: (0, 0)),
        ],
        out_specs=pl.BlockSpec((_BN, D), lambda i: (i, 0)),
        out_shape=jax.ShapeDtypeStruct((N, D), jnp.float32),
    )(x, s, pagg, W, b2)


def kernel(entity_embed, edge_index, W, b):
    src = edge_index[0]
    dst = edge_index[1]
    src2d = jnp.concatenate(
        [src, jnp.zeros((EPAD - E,), jnp.int32)]).reshape(NW, CPT, CHUNK)
    dst2d = jnp.concatenate(
        [dst, jnp.full((EPAD - E,), N, jnp.int32)]).reshape(NW, CPT, CHUNK)
    partials = _deg_kernel(dst2d)
    node, s = _scale_call(partials[:, :N], entity_embed)
    pagg = _edge_kernel(src2d, dst2d, node)
    return _final_call(entity_embed, s, pagg, W, b.reshape(1, D))


# speedup vs baseline: 4.1062x; 1.0442x over previous
import dataclasses
import functools

import jax
import jax.numpy as jnp
from jax import lax
from jax.experimental import pallas as pl
from jax.experimental.pallas import tpu as pltpu
from jax.experimental.pallas import tpu_sc as plsc

N = 10000
E = 320000
D = 128
NC = 2
NS = 16
NW = NC * NS
CHUNK = 128
BLK = 8
CPT = 80
NBLK = CPT // BLK
EPT = CPT * CHUNK
EPAD = EPT * NW
NPAD = 10112
RPT = NPAD // NS

_mesh = plsc.VectorSubcoreMesh(core_axis_name="c", subcore_axis_name="s")

_sc_params = pltpu.CompilerParams()
if "needs_layout_passes" in pltpu.CompilerParams.__dataclass_fields__:
    _sc_params = dataclasses.replace(_sc_params, needs_layout_passes=False)


@functools.partial(
    pl.kernel,
    out_type=jax.ShapeDtypeStruct((NW, NPAD), jnp.float32),
    mesh=_mesh,
    scratch_types=[
        pltpu.VMEM((CPT, CHUNK), jnp.int32),
        pltpu.VMEM((NPAD,), jnp.float32),
    ],
    compiler_params=_sc_params,
)
def _deg_kernel(dst_hbm, out_hbm, idx_v, cnt_v):
    wid = lax.axis_index("s") * NC + lax.axis_index("c")
    zero16 = jnp.zeros((16,), jnp.float32)

    @pl.loop(0, NPAD, step=16)
    def _(i):
        cnt_v[pl.ds(i, 16)] = zero16

    pltpu.sync_copy(dst_hbm.at[wid], idx_v)
    ones16 = jnp.ones((16,), jnp.float32)

    @pl.loop(0, CPT)
    def _(j):
        @pl.loop(0, CHUNK, step=16)
        def _(k):
            plsc.addupdate_scatter(cnt_v, [idx_v[j, pl.ds(k, 16)]], ones16)

    pltpu.sync_copy(cnt_v, out_hbm.at[wid])


@functools.partial(
    pl.kernel,
    out_type=jax.ShapeDtypeStruct((NC, NPAD, D), jnp.float32),
    mesh=_mesh,
    scratch_types=[
        pltpu.VMEM((BLK, CHUNK), jnp.int32),
        pltpu.VMEM((BLK, CHUNK), jnp.int32),
        pltpu.VMEM((BLK, CHUNK), jnp.int32),
        pltpu.VMEM((BLK, CHUNK), jnp.int32),
        pltpu.VMEM((CHUNK, D), jnp.float32),
        pltpu.VMEM((CHUNK, D), jnp.float32),
        pltpu.VMEM_SHARED((NPAD, D), jnp.float32),
        pltpu.SemaphoreType.DMA,
        pltpu.SemaphoreType.DMA,
        pltpu.SemaphoreType.DMA,
        pltpu.SemaphoreType.DMA,
        pltpu.SemaphoreType.DMA,
        pltpu.SemaphoreType.DMA,
    ],
    compiler_params=_sc_params,
)
def _edge_kernel(src_hbm, dst_hbm, node_hbm, out_hbm,
                 srcA, dstA, srcB, dstB, rows0, rows1, acc,
                 isemA, isemB, gsem0, gsem1, ssem0, ssem1):
    c = lax.axis_index("c")
    s = lax.axis_index("s")
    wid = s * NC + c
    zero16 = jnp.zeros((16,), jnp.float32)

    @pl.loop(0, CHUNK)
    def _(r):
        @pl.loop(0, D, step=16)
        def _(col):
            rows0[r, pl.ds(col, 16)] = zero16

    base = s * RPT
    for t in range(RPT // CHUNK):
        pltpu.sync_copy(rows0, acc.at[pl.ds(base + t * CHUNK, CHUNK)])
    if RPT % CHUNK:
        pltpu.sync_copy(rows0.at[pl.ds(0, RPT % CHUNK)],
                        acc.at[pl.ds(base + (RPT // CHUNK) * CHUNK, RPT % CHUNK)])

    plsc.subcore_barrier()

    idxp = [(srcA, dstA, isemA), (srcB, dstB, isemB)]
    rowp = [(rows0, gsem0, ssem0), (rows1, gsem1, ssem1)]

    def load_idx(b):
        sb, db, isem = idxp[b % 2]
        return (pltpu.async_copy(src_hbm.at[wid, pl.ds(b * BLK, BLK)], sb, isem),
                pltpu.async_copy(dst_hbm.at[wid, pl.ds(b * BLK, BLK)], db, isem))

    iA = load_idx(0)
    iA[0].wait()
    iA[1].wait()
    idesc = {1: load_idx(1)} if NBLK > 1 else {}
    gdesc = {0: pltpu.async_copy(node_hbm.at[srcA.at[0]], rows0, gsem0)}
    sdesc = {}
    for g in range(CPT):
        b = g // BLK
        jj = g % BLK
        buf, _, ssem = rowp[g % 2]
        gdesc[g].wait()
        sdesc[g] = pltpu.async_copy(buf, acc.at[idxp[b % 2][1].at[jj]],
                                    ssem, add=True)
        if g >= 1:
            sdesc[g - 1].wait()
        if jj == 0 and b + 1 < NBLK and b >= 1:
            idesc[b + 1] = load_idx(b + 1)
        gn = g + 1
        if gn < CPT:
            bn = gn // BLK
            if gn % BLK == 0:
                idesc[bn][0].wait()
                idesc[bn][1].wait()
            nbuf, gsem, _ = rowp[gn % 2]
            gdesc[gn] = pltpu.async_copy(
                node_hbm.at[idxp[bn % 2][0].at[gn % BLK]], nbuf, gsem)
    sdesc[CPT - 1].wait()

    plsc.subcore_barrier()
    pltpu.sync_copy(acc.at[pl.ds(base, RPT)], out_hbm.at[c, pl.ds(base, RPT)])


_BN = 1000


def _scale_body(p_ref, x_ref, node_ref, s_ref):
    deg = jnp.sum(p_ref[...], axis=0)
    s = 1.0 / jnp.sqrt(jnp.maximum(deg, 1.0))
    s_ref[...] = s[:, None]
    node_ref[...] = x_ref[...] * s[:, None]


def _scale_call(partials, x):
    return pl.pallas_call(
        _scale_body,
        out_shape=[
            jax.ShapeDtypeStruct((N, D), jnp.float32),
            jax.ShapeDtypeStruct((N, 1), jnp.float32),
        ],
    )(partials, x)


def _final_body(x_ref, s_ref, p_ref, w_ref, b_ref, o_ref):
    agg = p_ref[0] + p_ref[1]
    h = x_ref[...] + agg * s_ref[...]
    z = lax.dot_general(h, w_ref[...], (((1,), (1,)), ((), ())),
                        preferred_element_type=jnp.float32) + b_ref[...]
    o_ref[...] = jnp.where(z >= 0, z, 0.01 * z)


def _final_call(x, s, pagg, W, b2):
    return pl.pallas_call(
        _final_body,
        grid=(N // _BN,),
        in_specs=[
            pl.BlockSpec((_BN, D), lambda i: (i, 0)),
            pl.BlockSpec((_BN, 1), lambda i: (i, 0)),
            pl.BlockSpec((NC, _BN, D), lambda i: (0, i, 0)),
            pl.BlockSpec((D, D), lambda i: (0, 0)),
            pl.BlockSpec((1, D), lambda i: (0, 0)),
        ],
        out_specs=pl.BlockSpec((_BN, D), lambda i: (i, 0)),
        out_shape=jax.ShapeDtypeStruct((N, D), jnp.float32),
    )(x, s, pagg, W, b2)


def kernel(entity_embed, edge_index, W, b):
    src = edge_index[0]
    dst = edge_index[1]
    src2d = jnp.concatenate(
        [src, jnp.zeros((EPAD - E,), jnp.int32)]).reshape(NW, CPT, CHUNK)
    dst2d = jnp.concatenate(
        [dst, jnp.full((EPAD - E,), N, jnp.int32)]).reshape(NW, CPT, CHUNK)
    partials = _deg_kernel(dst2d)
    node, s = _scale_call(partials[:, :N], entity_embed)
    pagg = _edge_kernel(src2d, dst2d, node)
    return _final_call(entity_embed, s, pagg, W, b.reshape(1, D))


# speedup vs baseline: 4.2246x; 1.0288x over previous
import dataclasses
import functools

import jax
import jax.numpy as jnp
from jax import lax
from jax.experimental import pallas as pl
from jax.experimental.pallas import tpu as pltpu
from jax.experimental.pallas import tpu_sc as plsc

N = 10000
E = 320000
D = 128
NC = 2
NS = 16
NW = NC * NS
CHUNK = 128
BLK = 2
CPT = 80
NBLK = CPT // BLK
EPT = CPT * CHUNK
EPAD = EPT * NW
NPAD = 10016
STRIDE = 624
DUMP = 656

_mesh = plsc.VectorSubcoreMesh(core_axis_name="c", subcore_axis_name="s")

_sc_params = pltpu.CompilerParams()
if "needs_layout_passes" in pltpu.CompilerParams.__dataclass_fields__:
    _sc_params = dataclasses.replace(_sc_params, needs_layout_passes=False)


@functools.partial(
    pl.kernel,
    out_type=jax.ShapeDtypeStruct((NW, NPAD), jnp.float32),
    mesh=_mesh,
    scratch_types=[
        pltpu.VMEM((CPT, CHUNK), jnp.int32),
        pltpu.VMEM((NPAD,), jnp.float32),
    ],
    compiler_params=_sc_params,
)
def _deg_kernel(dst_hbm, out_hbm, idx_v, cnt_v):
    wid = lax.axis_index("s") * NC + lax.axis_index("c")
    zero16 = jnp.zeros((16,), jnp.float32)

    @pl.loop(0, NPAD, step=16)
    def _(i):
        cnt_v[pl.ds(i, 16)] = zero16

    pltpu.sync_copy(dst_hbm.at[wid], idx_v)
    ones16 = jnp.ones((16,), jnp.float32)

    @pl.loop(0, CPT)
    def _(j):
        @pl.loop(0, CHUNK, step=16)
        def _(k):
            plsc.addupdate_scatter(cnt_v, [idx_v[j, pl.ds(k, 16)]], ones16)

    pltpu.sync_copy(cnt_v, out_hbm.at[wid])


@functools.partial(
    pl.kernel,
    out_type=jax.ShapeDtypeStruct((NC, NPAD, D), jnp.float32),
    mesh=_mesh,
    scratch_types=[
        pltpu.VMEM((BLK, CHUNK), jnp.int32),
        pltpu.VMEM((BLK, CHUNK), jnp.int32),
        pltpu.VMEM((BLK, CHUNK), jnp.int32),
        pltpu.VMEM((BLK, CHUNK), jnp.int32),
        pltpu.VMEM((BLK, CHUNK), jnp.int32),
        pltpu.VMEM((BLK, CHUNK), jnp.int32),
        pltpu.VMEM((CHUNK, D), jnp.float32),
        pltpu.VMEM((CHUNK, D), jnp.float32),
        pltpu.VMEM((CHUNK, D), jnp.float32),
        pltpu.VMEM_SHARED((NPAD, D), jnp.float32),
        pltpu.SemaphoreType.DMA,
        pltpu.SemaphoreType.DMA,
        pltpu.SemaphoreType.DMA,
        pltpu.SemaphoreType.DMA,
        pltpu.SemaphoreType.DMA,
        pltpu.SemaphoreType.DMA,
        pltpu.SemaphoreType.DMA,
        pltpu.SemaphoreType.DMA,
        pltpu.SemaphoreType.DMA,
    ],
    compiler_params=_sc_params,
)
def _edge_kernel(src_hbm, dst_hbm, node_hbm, out_hbm,
                 srcA, dstA, srcB, dstB, srcC, dstC, rows0, rows1, rows2, acc,
                 isemA, isemB, isemC, gsem0, gsem1, gsem2,
                 ssem0, ssem1, ssem2):
    c = lax.axis_index("c")
    s = lax.axis_index("s")
    wid = s * NC + c
    zero16 = jnp.zeros((16,), jnp.float32)

    @pl.loop(0, CHUNK)
    def _(r):
        @pl.loop(0, D, step=16)
        def _(col):
            rows0[r, pl.ds(col, 16)] = zero16

    base = s * STRIDE
    for t in range(DUMP // CHUNK):
        pltpu.sync_copy(rows0, acc.at[pl.ds(base + t * CHUNK, CHUNK)])
    if DUMP % CHUNK:
        pltpu.sync_copy(rows0.at[pl.ds(0, DUMP % CHUNK)],
                        acc.at[pl.ds(base + (DUMP // CHUNK) * CHUNK, DUMP % CHUNK)])

    plsc.subcore_barrier()

    idxp = [(srcA, dstA, isemA), (srcB, dstB, isemB), (srcC, dstC, isemC)]
    rowp = [(rows0, gsem0, ssem0), (rows1, gsem1, ssem1),
            (rows2, gsem2, ssem2)]

    def load_idx(b):
        sb, db, isem = idxp[b % 3]
        return (pltpu.async_copy(src_hbm.at[wid, pl.ds(b * BLK, BLK)], sb, isem),
                pltpu.async_copy(dst_hbm.at[wid, pl.ds(b * BLK, BLK)], db, isem))

    def gather(h):
        bh, jh = divmod(h, BLK)
        buf, gsem, _ = rowp[h % 3]
        return pltpu.async_copy(node_hbm.at[idxp[bh % 3][0].at[jh]], buf, gsem)

    idesc = {0: load_idx(0)}
    idesc[0][0].wait()
    idesc[0][1].wait()
    waited = {0}
    if NBLK > 1:
        idesc[1] = load_idx(1)
    gdesc = {0: gather(0), 1: gather(1)}
    sdesc = {}
    for g in range(CPT):
        b, jj = divmod(g, BLK)
        buf, _, ssem = rowp[g % 3]
        gdesc[g].wait()
        sdesc[g] = pltpu.async_copy(buf, acc.at[idxp[b % 3][1].at[jj]],
                                    ssem, add=True)
        h = g + 2
        if h < CPT:
            if g >= 1:
                sdesc[g - 1].wait()
            bh = h // BLK
            if bh not in waited:
                idesc[bh][0].wait()
                idesc[bh][1].wait()
                waited.add(bh)
                if bh + 1 < NBLK:
                    idesc[bh + 1] = load_idx(bh + 1)
            gdesc[h] = gather(h)
    sdesc[CPT - 3].wait()
    sdesc[CPT - 2].wait()
    sdesc[CPT - 1].wait()

    plsc.subcore_barrier()
    pltpu.sync_copy(acc.at[pl.ds(base, DUMP)], out_hbm.at[c, pl.ds(base, DUMP)])


_BN = 1000


def _scale_body(p_ref, x_ref, node_ref, s_ref):
    deg = jnp.sum(p_ref[...], axis=0)
    s = 1.0 / jnp.sqrt(jnp.maximum(deg, 1.0))
    s_ref[...] = s[:, None]
    node_ref[...] = x_ref[...] * s[:, None]


def _scale_call(partials, x):
    return pl.pallas_call(
        _scale_body,
        out_shape=[
            jax.ShapeDtypeStruct((N, D), jnp.float32),
            jax.ShapeDtypeStruct((N, 1), jnp.float32),
        ],
    )(partials, x)


def _final_body(x_ref, s_ref, p_ref, w_ref, b_ref, o_ref):
    agg = p_ref[0] + p_ref[1]
    h = x_ref[...] + agg * s_ref[...]
    z = lax.dot_general(h, w_ref[...], (((1,), (1,)), ((), ())),
                        preferred_element_type=jnp.float32) + b_ref[...]
    o_ref[...] = jnp.where(z >= 0, z, 0.01 * z)


def _final_call(x, s, pagg, W, b2):
    return pl.pallas_call(
        _final_body,
        grid=(N // _BN,),
        in_specs=[
            pl.BlockSpec((_BN, D), lambda i: (i, 0)),
            pl.BlockSpec((_BN, 1), lambda i: (i, 0)),
            pl.BlockSpec((NC, _BN, D), lambda i: (0, i, 0)),
            pl.BlockSpec((D, D), lambda i: (0, 0)),
            pl.BlockSpec((1, D), lambda i: (0, 0)),
        ],
        out_specs=pl.BlockSpec((_BN, D), lambda i: (i, 0)),
        out_shape=jax.ShapeDtypeStruct((N, D), jnp.float32),
    )(x, s, pagg, W, b2)


def kernel(entity_embed, edge_index, W, b):
    src = edge_index[0]
    dst = edge_index[1]
    src2d = jnp.concatenate(
        [src, jnp.zeros((EPAD - E,), jnp.int32)]).reshape(NW, CPT, CHUNK)
    dst2d = jnp.concatenate(
        [dst, jnp.full((EPAD - E,), N, jnp.int32)]).reshape(NW, CPT, CHUNK)
    partials = _deg_kernel(dst2d)
    node, s = _scale_call(partials[:, :N], entity_embed)
    pagg = _edge_kernel(src2d, dst2d, node)
    return _final_call(entity_embed, s, pagg, W, b.reshape(1, D))


# speedup vs baseline: 4.9177x; 1.1641x over previous
import dataclasses
import functools

import jax
import jax.numpy as jnp
from jax import lax
from jax.experimental import pallas as pl
from jax.experimental.pallas import tpu as pltpu
from jax.experimental.pallas import tpu_sc as plsc

N = 10000
E = 320000
D = 128
NC = 2
NS = 16
NW = NC * NS
CHUNK = 128
BLK = 2
CPT = 80
NBLK = CPT // BLK
EPT = CPT * CHUNK
EPAD = EPT * NW
CPT0 = 120
CPT1 = 40
NPAD = 10016
STRIDE = 624
DUMP = 656

_mesh = plsc.VectorSubcoreMesh(core_axis_name="c", subcore_axis_name="s")

_sc_params = pltpu.CompilerParams()
if "needs_layout_passes" in pltpu.CompilerParams.__dataclass_fields__:
    _sc_params = dataclasses.replace(_sc_params, needs_layout_passes=False)


@functools.partial(
    pl.kernel,
    out_type=jax.ShapeDtypeStruct((NW, NPAD), jnp.float32),
    mesh=_mesh,
    scratch_types=[
        pltpu.VMEM((CPT, CHUNK), jnp.int32),
        pltpu.VMEM((NPAD,), jnp.float32),
    ],
    compiler_params=_sc_params,
)
def _deg_kernel(dst_hbm, out_hbm, idx_v, cnt_v):
    wid = lax.axis_index("s") * NC + lax.axis_index("c")
    zero16 = jnp.zeros((16,), jnp.float32)

    @pl.loop(0, NPAD, step=16)
    def _(i):
        cnt_v[pl.ds(i, 16)] = zero16

    pltpu.sync_copy(dst_hbm.at[wid], idx_v)
    ones16 = jnp.ones((16,), jnp.float32)

    @pl.loop(0, CPT)
    def _(j):
        @pl.loop(0, CHUNK, step=16)
        def _(k):
            plsc.addupdate_scatter(cnt_v, [idx_v[j, pl.ds(k, 16)]], ones16)

    pltpu.sync_copy(cnt_v, out_hbm.at[wid])


@functools.partial(
    pl.kernel,
    out_type=jax.ShapeDtypeStruct((NC, NPAD, D), jnp.float32),
    mesh=_mesh,
    scratch_types=[
        pltpu.VMEM((BLK, CHUNK), jnp.int32),
        pltpu.VMEM((BLK, CHUNK), jnp.int32),
        pltpu.VMEM((BLK, CHUNK), jnp.int32),
        pltpu.VMEM((BLK, CHUNK), jnp.int32),
        pltpu.VMEM((BLK, CHUNK), jnp.int32),
        pltpu.VMEM((BLK, CHUNK), jnp.int32),
        pltpu.VMEM((CHUNK, D), jnp.float32),
        pltpu.VMEM((CHUNK, D), jnp.float32),
        pltpu.VMEM((CHUNK, D), jnp.float32),
        pltpu.VMEM_SHARED((NPAD, D), jnp.float32),
        pltpu.SemaphoreType.DMA,
        pltpu.SemaphoreType.DMA,
        pltpu.SemaphoreType.DMA,
        pltpu.SemaphoreType.DMA,
        pltpu.SemaphoreType.DMA,
        pltpu.SemaphoreType.DMA,
        pltpu.SemaphoreType.DMA,
        pltpu.SemaphoreType.DMA,
        pltpu.SemaphoreType.DMA,
    ],
    compiler_params=_sc_params,
)
def _edge_kernel(src0_hbm, dst0_hbm, src1_hbm, dst1_hbm, node_hbm, out_hbm,
                 srcA, dstA, srcB, dstB, srcC, dstC, rows0, rows1, rows2, acc,
                 isemA, isemB, isemC, gsem0, gsem1, gsem2,
                 ssem0, ssem1, ssem2):
    c = lax.axis_index("c")
    s = lax.axis_index("s")
    zero16 = jnp.zeros((16,), jnp.float32)

    @pl.loop(0, CHUNK)
    def _(r):
        @pl.loop(0, D, step=16)
        def _(col):
            rows0[r, pl.ds(col, 16)] = zero16

    base = s * STRIDE
    for t in range(DUMP // CHUNK):
        pltpu.sync_copy(rows0, acc.at[pl.ds(base + t * CHUNK, CHUNK)])
    if DUMP % CHUNK:
        pltpu.sync_copy(rows0.at[pl.ds(0, DUMP % CHUNK)],
                        acc.at[pl.ds(base + (DUMP // CHUNK) * CHUNK, DUMP % CHUNK)])

    plsc.subcore_barrier()

    idxp = [(srcA, dstA, isemA), (srcB, dstB, isemB), (srcC, dstC, isemC)]
    rowp = [(rows0, gsem0, ssem0), (rows1, gsem1, ssem1),
            (rows2, gsem2, ssem2)]

    def pipeline(src_hbm, dst_hbm, cpt):
        nblk = cpt // BLK

        def load_idx(b):
            sb, db, isem = idxp[b % 3]
            return (pltpu.async_copy(src_hbm.at[s, pl.ds(b * BLK, BLK)],
                                     sb, isem),
                    pltpu.async_copy(dst_hbm.at[s, pl.ds(b * BLK, BLK)],
                                     db, isem))

        def gather(h):
            bh, jh = divmod(h, BLK)
            buf, gsem, _ = rowp[h % 3]
            return pltpu.async_copy(node_hbm.at[idxp[bh % 3][0].at[jh]],
                                    buf, gsem)

        idesc = {0: load_idx(0)}
        idesc[0][0].wait()
        idesc[0][1].wait()
        waited = {0}
        if nblk > 1:
            idesc[1] = load_idx(1)
        gdesc = {0: gather(0), 1: gather(1)}
        sdesc = {}
        for g in range(cpt):
            b, jj = divmod(g, BLK)
            buf, _, ssem = rowp[g % 3]
            gdesc[g].wait()
            sdesc[g] = pltpu.async_copy(buf, acc.at[idxp[b % 3][1].at[jj]],
                                        ssem, add=True)
            h = g + 2
            if h < cpt:
                if g >= 1:
                    sdesc[g - 1].wait()
                bh = h // BLK
                if bh not in waited:
                    idesc[bh][0].wait()
                    idesc[bh][1].wait()
                    waited.add(bh)
                    if bh + 1 < nblk:
                        idesc[bh + 1] = load_idx(bh + 1)
                gdesc[h] = gather(h)
        sdesc[cpt - 3].wait()
        sdesc[cpt - 2].wait()
        sdesc[cpt - 1].wait()

    @pl.when(c == 0)
    def _():
        pipeline(src0_hbm, dst0_hbm, CPT0)

    @pl.when(c == 1)
    def _():
        pipeline(src1_hbm, dst1_hbm, CPT1)

    plsc.subcore_barrier()
    pltpu.sync_copy(acc.at[pl.ds(base, DUMP)], out_hbm.at[c, pl.ds(base, DUMP)])


_BN = 1000


def _scale_body(p_ref, x_ref, node_ref, s_ref):
    deg = jnp.sum(p_ref[...], axis=0)
    s = 1.0 / jnp.sqrt(jnp.maximum(deg, 1.0))
    s_ref[...] = s[:, None]
    node_ref[...] = x_ref[...] * s[:, None]


def _scale_call(partials, x):
    return pl.pallas_call(
        _scale_body,
        out_shape=[
            jax.ShapeDtypeStruct((N, D), jnp.float32),
            jax.ShapeDtypeStruct((N, 1), jnp.float32),
        ],
    )(partials, x)


def _final_body(x_ref, s_ref, p_ref, w_ref, b_ref, o_ref):
    agg = p_ref[0] + p_ref[1]
    h = x_ref[...] + agg * s_ref[...]
    z = lax.dot_general(h, w_ref[...], (((1,), (1,)), ((), ())),
                        preferred_element_type=jnp.float32) + b_ref[...]
    o_ref[...] = jnp.where(z >= 0, z, 0.01 * z)


def _final_call(x, s, pagg, W, b2):
    return pl.pallas_call(
        _final_body,
        grid=(N // _BN,),
        in_specs=[
            pl.BlockSpec((_BN, D), lambda i: (i, 0)),
            pl.BlockSpec((_BN, 1), lambda i: (i, 0)),
            pl.BlockSpec((NC, _BN, D), lambda i: (0, i, 0)),
            pl.BlockSpec((D, D), lambda i: (0, 0)),
            pl.BlockSpec((1, D), lambda i: (0, 0)),
        ],
        out_specs=pl.BlockSpec((_BN, D), lambda i: (i, 0)),
        out_shape=jax.ShapeDtypeStruct((N, D), jnp.float32),
    )(x, s, pagg, W, b2)


def kernel(entity_embed, edge_index, W, b):
    src = edge_index[0]
    dst = edge_index[1]
    srcp = jnp.concatenate([src, jnp.zeros((EPAD - E,), jnp.int32)])
    dstp = jnp.concatenate([dst, jnp.full((EPAD - E,), N, jnp.int32)])
    dst2d = dstp.reshape(NW, CPT, CHUNK)
    src3 = srcp.reshape(NS, CPT0 + CPT1, CHUNK)
    dst3 = dstp.reshape(NS, CPT0 + CPT1, CHUNK)
    partials = _deg_kernel(dst2d)
    node, s = _scale_call(partials[:, :N], entity_embed)
    pagg = _edge_kernel(src3[:, :CPT0], dst3[:, :CPT0],
                        src3[:, CPT0:], dst3[:, CPT0:], node)
    return _final_call(entity_embed, s, pagg, W, b.reshape(1, D))


# speedup vs baseline: 4.9990x; 1.0165x over previous
import dataclasses
import functools

import jax
import jax.numpy as jnp
from jax import lax
from jax.experimental import pallas as pl
from jax.experimental.pallas import tpu as pltpu
from jax.experimental.pallas import tpu_sc as plsc

N = 10000
E = 320000
D = 128
NC = 2
NS = 16
NW = NC * NS
CHUNK = 128
BLK = 2
CPT = 80
NBLK = CPT // BLK
EPT = CPT * CHUNK
EPAD = EPT * NW
CPT0 = 124
CPT1 = 36
NPAD = 10016
STRIDE = 624
DUMP = 656

_mesh = plsc.VectorSubcoreMesh(core_axis_name="c", subcore_axis_name="s")

_sc_params = pltpu.CompilerParams()
if "needs_layout_passes" in pltpu.CompilerParams.__dataclass_fields__:
    _sc_params = dataclasses.replace(_sc_params, needs_layout_passes=False)


@functools.partial(
    pl.kernel,
    out_type=jax.ShapeDtypeStruct((NW, NPAD), jnp.float32),
    mesh=_mesh,
    scratch_types=[
        pltpu.VMEM((CPT, CHUNK), jnp.int32),
        pltpu.VMEM((NPAD,), jnp.float32),
    ],
    compiler_params=_sc_params,
)
def _deg_kernel(dst_hbm, out_hbm, idx_v, cnt_v):
    wid = lax.axis_index("s") * NC + lax.axis_index("c")
    zero16 = jnp.zeros((16,), jnp.float32)

    @pl.loop(0, NPAD, step=16)
    def _(i):
        cnt_v[pl.ds(i, 16)] = zero16

    pltpu.sync_copy(dst_hbm.at[wid], idx_v)
    ones16 = jnp.ones((16,), jnp.float32)

    @pl.loop(0, CPT)
    def _(j):
        @pl.loop(0, CHUNK, step=16)
        def _(k):
            plsc.addupdate_scatter(cnt_v, [idx_v[j, pl.ds(k, 16)]], ones16)

    pltpu.sync_copy(cnt_v, out_hbm.at[wid])


@functools.partial(
    pl.kernel,
    out_type=jax.ShapeDtypeStruct((NC, NPAD, D), jnp.float32),
    mesh=_mesh,
    scratch_types=[
        pltpu.VMEM((BLK, CHUNK), jnp.int32),
        pltpu.VMEM((BLK, CHUNK), jnp.int32),
        pltpu.VMEM((BLK, CHUNK), jnp.int32),
        pltpu.VMEM((BLK, CHUNK), jnp.int32),
        pltpu.VMEM((BLK, CHUNK), jnp.int32),
        pltpu.VMEM((BLK, CHUNK), jnp.int32),
        pltpu.VMEM((CHUNK, D), jnp.float32),
        pltpu.VMEM((CHUNK, D), jnp.float32),
        pltpu.VMEM((CHUNK, D), jnp.float32),
        pltpu.VMEM_SHARED((NPAD, D), jnp.float32),
        pltpu.SemaphoreType.DMA,
        pltpu.SemaphoreType.DMA,
        pltpu.SemaphoreType.DMA,
        pltpu.SemaphoreType.DMA,
        pltpu.SemaphoreType.DMA,
        pltpu.SemaphoreType.DMA,
        pltpu.SemaphoreType.DMA,
        pltpu.SemaphoreType.DMA,
        pltpu.SemaphoreType.DMA,
    ],
    compiler_params=_sc_params,
)
def _edge_kernel(src0_hbm, dst0_hbm, src1_hbm, dst1_hbm, node_hbm, out_hbm,
                 srcA, dstA, srcB, dstB, srcC, dstC, rows0, rows1, rows2, acc,
                 isemA, isemB, isemC, gsem0, gsem1, gsem2,
                 ssem0, ssem1, ssem2):
    c = lax.axis_index("c")
    s = lax.axis_index("s")
    zero16 = jnp.zeros((16,), jnp.float32)

    @pl.loop(0, CHUNK)
    def _(r):
        @pl.loop(0, D, step=16)
        def _(col):
            rows0[r, pl.ds(col, 16)] = zero16

    base = s * STRIDE
    for t in range(DUMP // CHUNK):
        pltpu.sync_copy(rows0, acc.at[pl.ds(base + t * CHUNK, CHUNK)])
    if DUMP % CHUNK:
        pltpu.sync_copy(rows0.at[pl.ds(0, DUMP % CHUNK)],
                        acc.at[pl.ds(base + (DUMP // CHUNK) * CHUNK, DUMP % CHUNK)])

    plsc.subcore_barrier()

    idxp = [(srcA, dstA, isemA), (srcB, dstB, isemB), (srcC, dstC, isemC)]
    rowp = [(rows0, gsem0, ssem0), (rows1, gsem1, ssem1),
            (rows2, gsem2, ssem2)]

    def pipeline(src_hbm, dst_hbm, cpt):
        nblk = cpt // BLK

        def load_idx(b):
            sb, db, isem = idxp[b % 3]
            return (pltpu.async_copy(src_hbm.at[s, pl.ds(b * BLK, BLK)],
                                     sb, isem),
                    pltpu.async_copy(dst_hbm.at[s, pl.ds(b * BLK, BLK)],
                                     db, isem))

        def gather(h):
            bh, jh = divmod(h, BLK)
            buf, gsem, _ = rowp[h % 3]
            return pltpu.async_copy(node_hbm.at[idxp[bh % 3][0].at[jh]],
                                    buf, gsem)

        idesc = {0: load_idx(0)}
        idesc[0][0].wait()
        idesc[0][1].wait()
        waited = {0}
        if nblk > 1:
            idesc[1] = load_idx(1)
        gdesc = {0: gather(0), 1: gather(1)}
        sdesc = {}
        for g in range(cpt):
            b, jj = divmod(g, BLK)
            buf, _, ssem = rowp[g % 3]
            gdesc[g].wait()
            sdesc[g] = pltpu.async_copy(buf, acc.at[idxp[b % 3][1].at[jj]],
                                        ssem, add=True)
            h = g + 2
            if h < cpt:
                if g >= 1:
                    sdesc[g - 1].wait()
                bh = h // BLK
                if bh not in waited:
                    idesc[bh][0].wait()
                    idesc[bh][1].wait()
                    waited.add(bh)
                    if bh + 1 < nblk:
                        idesc[bh + 1] = load_idx(bh + 1)
                gdesc[h] = gather(h)
        sdesc[cpt - 3].wait()
        sdesc[cpt - 2].wait()
        sdesc[cpt - 1].wait()

    @pl.when(c == 0)
    def _():
        pipeline(src0_hbm, dst0_hbm, CPT0)

    @pl.when(c == 1)
    def _():
        pipeline(src1_hbm, dst1_hbm, CPT1)

    plsc.subcore_barrier()
    pltpu.sync_copy(acc.at[pl.ds(base, DUMP)], out_hbm.at[c, pl.ds(base, DUMP)])


_BN = 1000


def _scale_body(p_ref, x_ref, node_ref, s_ref):
    deg = jnp.sum(p_ref[...], axis=0)
    s = 1.0 / jnp.sqrt(jnp.maximum(deg, 1.0))
    s_ref[...] = s[:, None]
    node_ref[...] = x_ref[...] * s[:, None]


def _scale_call(partials, x):
    return pl.pallas_call(
        _scale_body,
        out_shape=[
            jax.ShapeDtypeStruct((N, D), jnp.float32),
            jax.ShapeDtypeStruct((N, 1), jnp.float32),
        ],
    )(partials, x)


def _final_body(x_ref, s_ref, p_ref, w_ref, b_ref, o_ref):
    agg = p_ref[0] + p_ref[1]
    h = x_ref[...] + agg * s_ref[...]
    z = lax.dot_general(h, w_ref[...], (((1,), (1,)), ((), ())),
                        preferred_element_type=jnp.float32) + b_ref[...]
    o_ref[...] = jnp.where(z >= 0, z, 0.01 * z)


def _final_call(x, s, pagg, W, b2):
    return pl.pallas_call(
        _final_body,
        grid=(N // _BN,),
        in_specs=[
            pl.BlockSpec((_BN, D), lambda i: (i, 0)),
            pl.BlockSpec((_BN, 1), lambda i: (i, 0)),
            pl.BlockSpec((NC, _BN, D), lambda i: (0, i, 0)),
            pl.BlockSpec((D, D), lambda i: (0, 0)),
            pl.BlockSpec((1, D), lambda i: (0, 0)),
        ],
        out_specs=pl.BlockSpec((_BN, D), lambda i: (i, 0)),
        out_shape=jax.ShapeDtypeStruct((N, D), jnp.float32),
    )(x, s, pagg, W, b2)


def kernel(entity_embed, edge_index, W, b):
    src = edge_index[0]
    dst = edge_index[1]
    srcp = jnp.concatenate([src, jnp.zeros((EPAD - E,), jnp.int32)])
    dstp = jnp.concatenate([dst, jnp.full((EPAD - E,), N, jnp.int32)])
    dst2d = dstp.reshape(NW, CPT, CHUNK)
    src3 = srcp.reshape(NS, CPT0 + CPT1, CHUNK)
    dst3 = dstp.reshape(NS, CPT0 + CPT1, CHUNK)
    partials = _deg_kernel(dst2d)
    node, s = _scale_call(partials[:, :N], entity_embed)
    pagg = _edge_kernel(src3[:, :CPT0], dst3[:, :CPT0],
                        src3[:, CPT0:], dst3[:, CPT0:], node)
    return _final_call(entity_embed, s, pagg, W, b.reshape(1, D))
